# Initial kernel scaffold; baseline (speedup 1.0000x reference)
import functools

import jax
import jax.numpy as jnp
from jax import lax
from jax.experimental import pallas as pl
from jax.experimental.pallas import tpu as pltpu
from jax.experimental.pallas import tpu_sc as plsc

WEIGHT_ROWS = 4096
WEIGHT_COLS = 4096
CODE_DIM = 8
NUM_CODES = 8192
NUM_VECS = (WEIGHT_ROWS * WEIGHT_COLS) // CODE_DIM

NC, NS = 2, 16
NW = NC * NS
IDX_MINOR = 128
IDX_ROWS = NUM_VECS // IDX_MINOR
ROWS_PER_W = IDX_ROWS // NW
CH_ROWS = 16
N_CHUNKS = ROWS_PER_W // CH_ROWS
CH_VECS = CH_ROWS * IDX_MINOR


def _sc_gather_body(cb_hbm, idx_hbm, out_hbm, idx_v, rows_v, sem):
    wid = lax.axis_index("s") * NC + lax.axis_index("c")
    row0 = wid * ROWS_PER_W

    def chunk(g, carry):
        base_row = row0 + g * CH_ROWS
        pltpu.sync_copy(idx_hbm.at[pl.ds(base_row, CH_ROWS)], idx_v)
        handles = []
        for j in range(CH_ROWS):
            handles.append(
                pltpu.async_copy(
                    cb_hbm.at[idx_v.at[j]],
                    rows_v.at[pl.ds(j * IDX_MINOR, IDX_MINOR)],
                    sem,
                )
            )
        for h in handles:
            h.wait()
        out_base = base_row * IDX_MINOR
        pltpu.sync_copy(rows_v, out_hbm.at[pl.ds(out_base, CH_VECS)])
        return carry

    lax.fori_loop(0, N_CHUNKS, chunk, 0)


def _sc_gather(codebook, idx2d):
    mesh = plsc.VectorSubcoreMesh(core_axis_name="c", subcore_axis_name="s")
    fn = pl.kernel(
        _sc_gather_body,
        out_type=jax.ShapeDtypeStruct((NUM_VECS, CODE_DIM), jnp.float32),
        mesh=mesh,
        scratch_types=[
            pltpu.VMEM((CH_ROWS, IDX_MINOR), jnp.int32),
            pltpu.VMEM((CH_VECS, CODE_DIM), jnp.float32),
            pltpu.SemaphoreType.DMA,
        ],
    )
    return fn(codebook, idx2d)


BM, BN, BK = 512, 512, 512


def _mm_body(x_ref, g_ref, s_ref, o_ref):
    k = pl.program_id(2)

    @pl.when(k == 0)
    def _zero():
        o_ref[...] = jnp.zeros_like(o_ref)

    o_ref[...] += lax.dot_general(
        x_ref[...], g_ref[...], (((1,), (1,)), ((), ())),
        preferred_element_type=jnp.float32,
        precision=lax.Precision.HIGHEST,
    )

    @pl.when(k == pl.num_programs(2) - 1)
    def _scale():
        o_ref[...] *= s_ref[...]


def _tc_matmul(x, g, s_row):
    m, kdim = x.shape
    n = g.shape[0]
    grid = (m // BM, n // BN, kdim // BK)
    return pl.pallas_call(
        _mm_body,
        grid=grid,
        in_specs=[
            pl.BlockSpec((BM, BK), lambda i, j, k: (i, k)),
            pl.BlockSpec((BN, BK), lambda i, j, k: (j, k)),
            pl.BlockSpec((1, BN), lambda i, j, k: (0, j)),
        ],
        out_specs=pl.BlockSpec((BM, BN), lambda i, j, k: (i, j)),
        out_shape=jax.ShapeDtypeStruct((m, n), jnp.float32),
        compiler_params=pltpu.CompilerParams(
            dimension_semantics=("parallel", "parallel", "arbitrary"),
        ),
    )(x, g, s_row)


def kernel(x, indices, codebook, scales):
    idx2d = indices.reshape(IDX_ROWS, IDX_MINOR)
    g = _sc_gather(codebook, idx2d).reshape(WEIGHT_ROWS, WEIGHT_COLS)
    return _tc_matmul(x, g, scales.reshape(1, WEIGHT_ROWS))


# speedup vs baseline: 3.6205x; 3.6205x over previous
import functools

import jax
import jax.numpy as jnp
from jax import lax
from jax.experimental import pallas as pl
from jax.experimental.pallas import tpu as pltpu
from jax.experimental.pallas import tpu_sc as plsc

WEIGHT_ROWS = 4096
WEIGHT_COLS = 4096
CODE_DIM = 8
NUM_CODES = 8192
NUM_VECS = (WEIGHT_ROWS * WEIGHT_COLS) // CODE_DIM

NC, NS, L = 2, 16, 16
NW = NC * NS
VECS_PER_W = NUM_VECS // NW
CH = 2048
N_CHUNKS = VECS_PER_W // CH


def _sc_gather_body(cb_hbm, idx_hbm, out_hbm, cb_v, idx_v, rows_v):
    wid = lax.axis_index("s") * NC + lax.axis_index("c")
    base = wid * VECS_PER_W
    pltpu.sync_copy(cb_hbm, cb_v)
    lanes8 = lax.iota(jnp.int32, L) * CODE_DIM

    def chunk(g, carry):
        cbase = base + g * CH
        pltpu.sync_copy(idx_hbm.at[pl.ds(cbase, CH)], idx_v)

        def step(i, carry2):
            idx16 = idx_v[pl.ds(i * L, L)] * CODE_DIM
            outsel = lanes8 + i * (L * CODE_DIM)
            for d in range(CODE_DIM):
                vals = plsc.load_gather(cb_v, [idx16 + d])
                plsc.store_scatter(rows_v, [outsel + d], vals)
            return carry2

        lax.fori_loop(0, CH // L, step, 0)
        pltpu.sync_copy(rows_v, out_hbm.at[pl.ds(cbase * CODE_DIM, CH * CODE_DIM)])
        return carry

    lax.fori_loop(0, N_CHUNKS, chunk, 0)


def _sc_gather(codebook, indices):
    mesh = plsc.VectorSubcoreMesh(core_axis_name="c", subcore_axis_name="s")
    fn = pl.kernel(
        _sc_gather_body,
        out_type=jax.ShapeDtypeStruct((NUM_VECS * CODE_DIM,), jnp.float32),
        mesh=mesh,
        scratch_types=[
            pltpu.VMEM((NUM_CODES * CODE_DIM,), jnp.float32),
            pltpu.VMEM((CH,), jnp.int32),
            pltpu.VMEM((CH * CODE_DIM,), jnp.float32),
        ],
        compiler_params=pltpu.CompilerParams(needs_layout_passes=False),
    )
    return fn(codebook.reshape(-1), indices)


BM, BN, BK = 512, 512, 512


def _mm_body(x_ref, g_ref, s_ref, o_ref):
    k = pl.program_id(2)

    @pl.when(k == 0)
    def _zero():
        o_ref[...] = jnp.zeros_like(o_ref)

    o_ref[...] += lax.dot_general(
        x_ref[...], g_ref[...], (((1,), (1,)), ((), ())),
        preferred_element_type=jnp.float32,
        precision=lax.Precision.HIGHEST,
    )

    @pl.when(k == pl.num_programs(2) - 1)
    def _scale():
        o_ref[...] *= s_ref[...]


def _tc_matmul(x, g, s_row):
    m, kdim = x.shape
    n = g.shape[0]
    grid = (m // BM, n // BN, kdim // BK)
    return pl.pallas_call(
        _mm_body,
        grid=grid,
        in_specs=[
            pl.BlockSpec((BM, BK), lambda i, j, k: (i, k)),
            pl.BlockSpec((BN, BK), lambda i, j, k: (j, k)),
            pl.BlockSpec((1, BN), lambda i, j, k: (0, j)),
        ],
        out_specs=pl.BlockSpec((BM, BN), lambda i, j, k: (i, j)),
        out_shape=jax.ShapeDtypeStruct((m, n), jnp.float32),
        compiler_params=pltpu.CompilerParams(
            dimension_semantics=("parallel", "parallel", "arbitrary"),
        ),
    )(x, g, s_row)


def kernel(x, indices, codebook, scales):
    g = _sc_gather(codebook, indices).reshape(WEIGHT_ROWS, WEIGHT_COLS)
    return _tc_matmul(x, g, scales.reshape(1, WEIGHT_ROWS))


# speedup vs baseline: 5.6433x; 1.5587x over previous
import functools

import jax
import jax.numpy as jnp
from jax import lax
from jax.experimental import pallas as pl
from jax.experimental.pallas import tpu as pltpu
from jax.experimental.pallas import tpu_sc as plsc

WEIGHT_ROWS = 4096
WEIGHT_COLS = 4096
CODE_DIM = 8
NUM_CODES = 8192
NUM_VECS = (WEIGHT_ROWS * WEIGHT_COLS) // CODE_DIM

NC, NS, L = 2, 16, 16
NW = NC * NS
VECS_PER_W = NUM_VECS // NW
CH = 2048
N_CHUNKS = VECS_PER_W // CH


def _sc_gather_body(cb_hbm, idx_hbm, out_hbm, cb_v, idx_v, rows_v):
    wid = lax.axis_index("s") * NC + lax.axis_index("c")
    base = wid * VECS_PER_W
    pltpu.sync_copy(cb_hbm, cb_v)
    lanes8 = lax.iota(jnp.int32, L) * CODE_DIM

    def chunk(g, carry):
        cbase = base + g * CH
        pltpu.sync_copy(idx_hbm.at[pl.ds(cbase, CH)], idx_v)

        def step(i, carry2):
            idx16 = idx_v[pl.ds(i * L, L)] * CODE_DIM
            outsel = lanes8 + i * (L * CODE_DIM)
            for d in range(CODE_DIM):
                vals = plsc.load_gather(cb_v, [idx16 + d])
                plsc.store_scatter(rows_v, [outsel + d], vals)
            return carry2

        lax.fori_loop(0, CH // L, step, 0)
        pltpu.sync_copy(rows_v, out_hbm.at[pl.ds(cbase * CODE_DIM, CH * CODE_DIM)])
        return carry

    lax.fori_loop(0, N_CHUNKS, chunk, 0)


def _sc_gather(codebook, indices):
    mesh = plsc.VectorSubcoreMesh(core_axis_name="c", subcore_axis_name="s")
    fn = pl.kernel(
        _sc_gather_body,
        out_type=jax.ShapeDtypeStruct((NUM_VECS * CODE_DIM,), jnp.float32),
        mesh=mesh,
        scratch_types=[
            pltpu.VMEM((NUM_CODES * CODE_DIM,), jnp.float32),
            pltpu.VMEM((CH,), jnp.int32),
            pltpu.VMEM((CH * CODE_DIM,), jnp.float32),
        ],
        compiler_params=pltpu.CompilerParams(needs_layout_passes=False),
    )
    return fn(codebook.reshape(-1), indices)


BM, BN, BK = 512, 512, 512


def _mm_body(x_ref, g_ref, s_ref, o_ref):
    k = pl.program_id(2)

    @pl.when(k == 0)
    def _zero():
        o_ref[...] = jnp.zeros_like(o_ref)

    o_ref[...] += lax.dot_general(
        x_ref[...], g_ref[...], (((1,), (1,)), ((), ())),
        preferred_element_type=jnp.float32,
        precision=lax.Precision.DEFAULT,
    )

    @pl.when(k == pl.num_programs(2) - 1)
    def _scale():
        o_ref[...] *= s_ref[...]


def _tc_matmul(x, g, s_row):
    m, kdim = x.shape
    n = g.shape[0]
    grid = (m // BM, n // BN, kdim // BK)
    return pl.pallas_call(
        _mm_body,
        grid=grid,
        in_specs=[
            pl.BlockSpec((BM, BK), lambda i, j, k: (i, k)),
            pl.BlockSpec((BN, BK), lambda i, j, k: (j, k)),
            pl.BlockSpec((1, BN), lambda i, j, k: (0, j)),
        ],
        out_specs=pl.BlockSpec((BM, BN), lambda i, j, k: (i, j)),
        out_shape=jax.ShapeDtypeStruct((m, n), jnp.float32),
        compiler_params=pltpu.CompilerParams(
            dimension_semantics=("parallel", "parallel", "arbitrary"),
        ),
    )(x, g, s_row)


def kernel(x, indices, codebook, scales):
    g = _sc_gather(codebook, indices).reshape(WEIGHT_ROWS, WEIGHT_COLS)
    return _tc_matmul(x, g, scales.reshape(1, WEIGHT_ROWS))


# speedup vs baseline: 5.8634x; 1.0390x over previous
import functools

import jax
import jax.numpy as jnp
from jax import lax
from jax.experimental import pallas as pl
from jax.experimental.pallas import tpu as pltpu
from jax.experimental.pallas import tpu_sc as plsc

WEIGHT_ROWS = 4096
WEIGHT_COLS = 4096
CODE_DIM = 8
NUM_CODES = 8192
NUM_VECS = (WEIGHT_ROWS * WEIGHT_COLS) // CODE_DIM

NC, NS, L = 2, 16, 16
NW = NC * NS
VECS_PER_W = NUM_VECS // NW
CH = 2048
N_CHUNKS = VECS_PER_W // CH
CODE_W = CODE_DIM // 2


def _sc_gather_body(cb_hbm, idx_hbm, out_hbm, cb_v, idx_v, rows_v):
    wid = lax.axis_index("s") * NC + lax.axis_index("c")
    base = wid * VECS_PER_W
    pltpu.sync_copy(cb_hbm, cb_v)
    lanes_w = lax.iota(jnp.int32, L) * CODE_W

    def chunk(g, carry):
        cbase = base + g * CH
        pltpu.sync_copy(idx_hbm.at[pl.ds(cbase, CH)], idx_v)

        def step(i, carry2):
            idx16 = idx_v[pl.ds(i * L, L)] * CODE_W
            outsel = lanes_w + i * (L * CODE_W)
            for d in range(CODE_W):
                vals = plsc.load_gather(cb_v, [idx16 + d])
                plsc.store_scatter(rows_v, [outsel + d], vals)
            return carry2

        lax.fori_loop(0, CH // L, step, 0)
        pltpu.sync_copy(rows_v, out_hbm.at[pl.ds(cbase * CODE_W, CH * CODE_W)])
        return carry

    lax.fori_loop(0, N_CHUNKS, chunk, 0)


def _sc_gather(cb_words, indices):
    mesh = plsc.VectorSubcoreMesh(core_axis_name="c", subcore_axis_name="s")
    fn = pl.kernel(
        _sc_gather_body,
        out_type=jax.ShapeDtypeStruct((NUM_VECS * CODE_W,), jnp.int32),
        mesh=mesh,
        scratch_types=[
            pltpu.VMEM((NUM_CODES * CODE_W,), jnp.int32),
            pltpu.VMEM((CH,), jnp.int32),
            pltpu.VMEM((CH * CODE_W,), jnp.int32),
        ],
        compiler_params=pltpu.CompilerParams(needs_layout_passes=False),
    )
    return fn(cb_words, indices)


BM, BN, BK = 512, 512, 512


def _mm_body(x_ref, g_ref, s_ref, o_ref):
    k = pl.program_id(2)

    @pl.when(k == 0)
    def _zero():
        o_ref[...] = jnp.zeros_like(o_ref)

    o_ref[...] += lax.dot_general(
        x_ref[...], g_ref[...], (((1,), (1,)), ((), ())),
        preferred_element_type=jnp.float32,
        precision=lax.Precision.DEFAULT,
    )

    @pl.when(k == pl.num_programs(2) - 1)
    def _scale():
        o_ref[...] *= s_ref[...]


def _tc_matmul(x, g, s_row):
    m, kdim = x.shape
    n = g.shape[0]
    grid = (m // BM, n // BN, kdim // BK)
    return pl.pallas_call(
        _mm_body,
        grid=grid,
        in_specs=[
            pl.BlockSpec((BM, BK), lambda i, j, k: (i, k)),
            pl.BlockSpec((BN, BK), lambda i, j, k: (j, k)),
            pl.BlockSpec((1, BN), lambda i, j, k: (0, j)),
        ],
        out_specs=pl.BlockSpec((BM, BN), lambda i, j, k: (i, j)),
        out_shape=jax.ShapeDtypeStruct((m, n), jnp.float32),
        compiler_params=pltpu.CompilerParams(
            dimension_semantics=("parallel", "parallel", "arbitrary"),
        ),
    )(x, g, s_row)


def kernel(x, indices, codebook, scales):
    cb_words = lax.bitcast_convert_type(
        codebook.astype(jnp.bfloat16).reshape(NUM_CODES, CODE_W, 2), jnp.int32
    ).reshape(-1)
    g_words = _sc_gather(cb_words, indices)
    g = lax.bitcast_convert_type(g_words, jnp.bfloat16).reshape(
        WEIGHT_ROWS, WEIGHT_COLS
    )
    return _tc_matmul(x.astype(jnp.bfloat16), g, scales.reshape(1, WEIGHT_ROWS))


# speedup vs baseline: 10.5565x; 1.8004x over previous
import functools

import jax
import jax.numpy as jnp
from jax import lax
from jax.experimental import pallas as pl
from jax.experimental.pallas import tpu as pltpu
from jax.experimental.pallas import tpu_sc as plsc

WEIGHT_ROWS = 4096
WEIGHT_COLS = 4096
CODE_DIM = 8
NUM_CODES = 8192
NUM_VECS = (WEIGHT_ROWS * WEIGHT_COLS) // CODE_DIM

NC, NS, L = 2, 16, 16
NW = NC * NS
VECS_PER_W = NUM_VECS // NW
CH = 2048
N_CHUNKS = VECS_PER_W // CH
CODE_W = CODE_DIM // 2


def _sc_gather_body(cb_hbm, idx_hbm, out_hbm, cb_v, idx_v, rows_v):
    wid = lax.axis_index("s") * NC + lax.axis_index("c")
    base = wid * VECS_PER_W
    pltpu.sync_copy(cb_hbm, cb_v)
    lanes_w = lax.iota(jnp.int32, L) * CODE_W

    def chunk(g, carry):
        cbase = base + g * CH
        pltpu.sync_copy(idx_hbm.at[pl.ds(cbase, CH)], idx_v)

        def step(i, carry2):
            idx16 = idx_v[pl.ds(i * L, L)] * CODE_W
            outsel = lanes_w + i * (L * CODE_W)
            for d in range(CODE_W):
                vals = plsc.load_gather(cb_v, [idx16 + d])
                plsc.store_scatter(rows_v, [outsel + d], vals)
            return carry2

        lax.fori_loop(0, CH // L, step, 0)
        pltpu.sync_copy(rows_v, out_hbm.at[pl.ds(cbase * CODE_W, CH * CODE_W)])
        return carry

    lax.fori_loop(0, N_CHUNKS, chunk, 0)


def _sc_gather(cb_words, indices):
    mesh = plsc.VectorSubcoreMesh(core_axis_name="c", subcore_axis_name="s")
    fn = pl.kernel(
        _sc_gather_body,
        out_type=jax.ShapeDtypeStruct((NUM_VECS * CODE_W,), jnp.int32),
        mesh=mesh,
        scratch_types=[
            pltpu.VMEM((NUM_CODES * CODE_W,), jnp.int32),
            pltpu.VMEM((CH,), jnp.int32),
            pltpu.VMEM((CH * CODE_W,), jnp.int32),
        ],
        compiler_params=pltpu.CompilerParams(needs_layout_passes=False),
    )
    return fn(cb_words, indices)


BM, BN, BK = 2048, 2048, 512


def _mm_body(x_ref, g_ref, s_ref, o_ref):
    k = pl.program_id(2)

    @pl.when(k == 0)
    def _zero():
        o_ref[...] = jnp.zeros_like(o_ref)

    o_ref[...] += lax.dot_general(
        x_ref[...], g_ref[...], (((1,), (1,)), ((), ())),
        preferred_element_type=jnp.float32,
        precision=lax.Precision.DEFAULT,
    )

    @pl.when(k == pl.num_programs(2) - 1)
    def _scale():
        o_ref[...] *= s_ref[...]


def _tc_matmul(x, g, s_row):
    m, kdim = x.shape
    n = g.shape[0]
    grid = (m // BM, n // BN, kdim // BK)
    return pl.pallas_call(
        _mm_body,
        grid=grid,
        in_specs=[
            pl.BlockSpec((BM, BK), lambda i, j, k: (i, k)),
            pl.BlockSpec((BN, BK), lambda i, j, k: (j, k)),
            pl.BlockSpec((1, BN), lambda i, j, k: (0, j)),
        ],
        out_specs=pl.BlockSpec((BM, BN), lambda i, j, k: (i, j)),
        out_shape=jax.ShapeDtypeStruct((m, n), jnp.float32),
        compiler_params=pltpu.CompilerParams(
            dimension_semantics=("parallel", "parallel", "arbitrary"),
        ),
    )(x, g, s_row)


def kernel(x, indices, codebook, scales):
    cb_words = lax.bitcast_convert_type(
        codebook.astype(jnp.bfloat16).reshape(NUM_CODES, CODE_W, 2), jnp.int32
    ).reshape(-1)
    g_words = _sc_gather(cb_words, indices)
    g = lax.bitcast_convert_type(g_words, jnp.bfloat16).reshape(
        WEIGHT_ROWS, WEIGHT_COLS
    )
    return _tc_matmul(x.astype(jnp.bfloat16), g, scales.reshape(1, WEIGHT_ROWS))


# speedup vs baseline: 10.7606x; 1.0193x over previous
import functools

import jax
import jax.numpy as jnp
from jax import lax
from jax.experimental import pallas as pl
from jax.experimental.pallas import tpu as pltpu
from jax.experimental.pallas import tpu_sc as plsc

WEIGHT_ROWS = 4096
WEIGHT_COLS = 4096
CODE_DIM = 8
NUM_CODES = 8192
NUM_VECS = (WEIGHT_ROWS * WEIGHT_COLS) // CODE_DIM

NC, NS, L = 2, 16, 16
NW = NC * NS
VECS_PER_W = NUM_VECS // NW
CH = 2048
N_CHUNKS = VECS_PER_W // CH
CODE_W = CODE_DIM // 2


def _sc_gather_body(cb_hbm, idx_hbm, out_hbm, cb_v, idx_v, rows_v):
    wid = lax.axis_index("s") * NC + lax.axis_index("c")
    base = wid * VECS_PER_W
    pltpu.sync_copy(cb_hbm, cb_v)
    lanes_w = lax.iota(jnp.int32, L) * CODE_W

    def chunk(g, carry):
        cbase = base + g * CH
        pltpu.sync_copy(idx_hbm.at[pl.ds(cbase, CH)], idx_v)

        def step(i, carry2):
            idx16 = idx_v[pl.ds(i * L, L)] * CODE_W
            outsel = lanes_w + i * (L * CODE_W)
            for d in range(CODE_W):
                vals = plsc.load_gather(cb_v, [idx16 + d])
                plsc.store_scatter(rows_v, [outsel + d], vals)
            return carry2

        lax.fori_loop(0, CH // L, step, 0)
        pltpu.sync_copy(rows_v, out_hbm.at[pl.ds(cbase * CODE_W, CH * CODE_W)])
        return carry

    lax.fori_loop(0, N_CHUNKS, chunk, 0)


def _sc_gather(cb_words, indices):
    mesh = plsc.VectorSubcoreMesh(core_axis_name="c", subcore_axis_name="s")
    fn = pl.kernel(
        _sc_gather_body,
        out_type=jax.ShapeDtypeStruct((NUM_VECS * CODE_W,), jnp.int32),
        mesh=mesh,
        scratch_types=[
            pltpu.VMEM((NUM_CODES * CODE_W,), jnp.int32),
            pltpu.VMEM((CH,), jnp.int32),
            pltpu.VMEM((CH * CODE_W,), jnp.int32),
        ],
        compiler_params=pltpu.CompilerParams(needs_layout_passes=False),
    )
    return fn(cb_words, indices)


BM, BN, BK = 2048, 2048, 1024


def _mm_body(x_ref, g_ref, s_ref, o_ref):
    k = pl.program_id(2)

    @pl.when(k == 0)
    def _zero():
        o_ref[...] = jnp.zeros_like(o_ref)

    o_ref[...] += lax.dot_general(
        x_ref[...], g_ref[...], (((1,), (1,)), ((), ())),
        preferred_element_type=jnp.float32,
        precision=lax.Precision.DEFAULT,
    )

    @pl.when(k == pl.num_programs(2) - 1)
    def _scale():
        o_ref[...] *= s_ref[...]


def _tc_matmul(x, g, s_row):
    m, kdim = x.shape
    n = g.shape[0]
    grid = (m // BM, n // BN, kdim // BK)
    return pl.pallas_call(
        _mm_body,
        grid=grid,
        in_specs=[
            pl.BlockSpec((BM, BK), lambda i, j, k: (i, k)),
            pl.BlockSpec((BN, BK), lambda i, j, k: (j, k)),
            pl.BlockSpec((1, BN), lambda i, j, k: (0, j)),
        ],
        out_specs=pl.BlockSpec((BM, BN), lambda i, j, k: (i, j)),
        out_shape=jax.ShapeDtypeStruct((m, n), jnp.float32),
        compiler_params=pltpu.CompilerParams(
            dimension_semantics=("parallel", "parallel", "arbitrary"),
        ),
    )(x, g, s_row)


def kernel(x, indices, codebook, scales):
    cb_words = lax.bitcast_convert_type(
        codebook.astype(jnp.bfloat16).reshape(NUM_CODES, CODE_W, 2), jnp.int32
    ).reshape(-1)
    g_words = _sc_gather(cb_words, indices)
    g = lax.bitcast_convert_type(g_words, jnp.bfloat16).reshape(
        WEIGHT_ROWS, WEIGHT_COLS
    )
    return _tc_matmul(x.astype(jnp.bfloat16), g, scales.reshape(1, WEIGHT_ROWS))


# speedup vs baseline: 10.9044x; 1.0134x over previous
import jax
import jax.numpy as jnp
from jax import lax
from jax.experimental import pallas as pl
from jax.experimental.pallas import tpu as pltpu
from jax.experimental.pallas import tpu_sc as plsc

WEIGHT_ROWS = 4096
WEIGHT_COLS = 4096
CODE_DIM = 8
NUM_CODES = 8192
NUM_VECS = (WEIGHT_ROWS * WEIGHT_COLS) // CODE_DIM

NC, NS, L = 2, 16, 16
NW = NC * NS
CH = 2048
CODE_W = CODE_DIM // 2

N_SPLIT = 2
VECS_PER_CALL = NUM_VECS // N_SPLIT
ROWS_PER_CALL = WEIGHT_ROWS // N_SPLIT


def _make_sc_gather_body(base_vec):
    vecs_per_w = VECS_PER_CALL // NW
    n_chunks = vecs_per_w // CH

    def body(cb_hbm, idx_hbm, out_hbm, cb_v, idx_v, rows_v):
        wid = lax.axis_index("s") * NC + lax.axis_index("c")
        base = base_vec + wid * vecs_per_w
        pltpu.sync_copy(cb_hbm, cb_v)
        lanes_w = lax.iota(jnp.int32, L) * CODE_W

        def chunk(g, carry):
            cbase = base + g * CH
            pltpu.sync_copy(idx_hbm.at[pl.ds(cbase, CH)], idx_v)

            def step(i, carry2):
                idx16 = idx_v[pl.ds(i * L, L)] * CODE_W
                outsel = lanes_w + i * (L * CODE_W)
                for d in range(CODE_W):
                    vals = plsc.load_gather(cb_v, [idx16 + d])
                    plsc.store_scatter(rows_v, [outsel + d], vals)
                return carry2

            lax.fori_loop(0, CH // L, step, 0)
            out_base = (cbase - base_vec) * CODE_W
            pltpu.sync_copy(rows_v, out_hbm.at[pl.ds(out_base, CH * CODE_W)])
            return carry

        lax.fori_loop(0, n_chunks, chunk, 0)

    return body


def _sc_gather(cb_words, indices, base_vec):
    mesh = plsc.VectorSubcoreMesh(core_axis_name="c", subcore_axis_name="s")
    fn = pl.kernel(
        _make_sc_gather_body(base_vec),
        out_type=jax.ShapeDtypeStruct((VECS_PER_CALL * CODE_W,), jnp.int32),
        mesh=mesh,
        scratch_types=[
            pltpu.VMEM((NUM_CODES * CODE_W,), jnp.int32),
            pltpu.VMEM((CH,), jnp.int32),
            pltpu.VMEM((CH * CODE_W,), jnp.int32),
        ],
        compiler_params=pltpu.CompilerParams(needs_layout_passes=False),
    )
    return fn(cb_words, indices)


BM, BK = 2048, 1024
BN = ROWS_PER_CALL


def _mm_compute(x_ref, g_ref, s_ref, o_ref):
    k = pl.program_id(1)

    @pl.when(k == 0)
    def _zero():
        o_ref[...] = jnp.zeros_like(o_ref)

    o_ref[...] += lax.dot_general(
        x_ref[...], g_ref[...], (((1,), (1,)), ((), ())),
        preferred_element_type=jnp.float32,
        precision=lax.Precision.DEFAULT,
    )

    @pl.when(k == pl.num_programs(1) - 1)
    def _scale():
        o_ref[...] *= s_ref[...]


def _mm_body_first(x_ref, g_ref, s_ref, o_ref):
    _mm_compute(x_ref, g_ref, s_ref, o_ref)


def _mm_body_rest(x_ref, g_ref, s_ref, prev_ref, o_ref):
    _mm_compute(x_ref, g_ref, s_ref, o_ref)


def _tc_matmul_part(x, g, s_row, prev, col):
    m, kdim = x.shape
    grid = (m // BM, kdim // BK)
    in_specs = [
        pl.BlockSpec((BM, BK), lambda i, k: (i, k)),
        pl.BlockSpec((BN, BK), lambda i, k: (0, k)),
        pl.BlockSpec((1, BN), lambda i, k: (0, 0)),
    ]
    args = (x, g, s_row)
    if prev is None:
        body, aliases = _mm_body_first, {}
    else:
        body, aliases = _mm_body_rest, {3: 0}
        in_specs.append(pl.BlockSpec(memory_space=pl.ANY))
        args = args + (prev,)
    return pl.pallas_call(
        body,
        grid=grid,
        in_specs=in_specs,
        out_specs=pl.BlockSpec((BM, BN), lambda i, k, c=col: (i, c)),
        out_shape=jax.ShapeDtypeStruct((m, WEIGHT_ROWS), jnp.float32),
        input_output_aliases=aliases,
        compiler_params=pltpu.CompilerParams(
            dimension_semantics=("parallel", "arbitrary"),
        ),
    )(*args)


def kernel(x, indices, codebook, scales):
    cb_words = lax.bitcast_convert_type(
        codebook.astype(jnp.bfloat16).reshape(NUM_CODES, CODE_W, 2), jnp.int32
    ).reshape(-1)
    x_bf = x.astype(jnp.bfloat16)
    s_row = scales.reshape(1, WEIGHT_ROWS)
    out = None
    for part in range(N_SPLIT):
        gw = _sc_gather(cb_words, indices, part * VECS_PER_CALL)
        g = lax.bitcast_convert_type(gw, jnp.bfloat16).reshape(
            ROWS_PER_CALL, WEIGHT_COLS
        )
        s_part = lax.slice_in_dim(s_row, part * BN, (part + 1) * BN, axis=1)
        out = _tc_matmul_part(x_bf, g, s_part, out, part)
    return out


# speedup vs baseline: 11.1415x; 1.0217x over previous
import jax
import jax.numpy as jnp
from jax import lax
from jax.experimental import pallas as pl
from jax.experimental.pallas import tpu as pltpu
from jax.experimental.pallas import tpu_sc as plsc

WEIGHT_ROWS = 4096
WEIGHT_COLS = 4096
CODE_DIM = 8
NUM_CODES = 8192
NUM_VECS = (WEIGHT_ROWS * WEIGHT_COLS) // CODE_DIM

NC, NS, L = 2, 16, 16
NW = NC * NS
CH = 8192
CODE_W = CODE_DIM // 2

N_SPLIT = 2
VECS_PER_CALL = NUM_VECS // N_SPLIT
ROWS_PER_CALL = WEIGHT_ROWS // N_SPLIT


def _make_sc_gather_body(base_vec):
    vecs_per_w = VECS_PER_CALL // NW
    n_chunks = vecs_per_w // CH

    def body(cb_hbm, idx_hbm, out_hbm, cb_v, idx_v, rows_v):
        wid = lax.axis_index("s") * NC + lax.axis_index("c")
        base = base_vec + wid * vecs_per_w
        pltpu.sync_copy(cb_hbm, cb_v)
        lanes_w = lax.iota(jnp.int32, L) * CODE_W

        def chunk(g, carry):
            cbase = base + g * CH
            pltpu.sync_copy(idx_hbm.at[pl.ds(cbase, CH)], idx_v)

            def step(i, carry2):
                idx16 = idx_v[pl.ds(i * L, L)] * CODE_W
                outsel = lanes_w + i * (L * CODE_W)
                for d in range(CODE_W):
                    vals = plsc.load_gather(cb_v, [idx16 + d])
                    plsc.store_scatter(rows_v, [outsel + d], vals)
                return carry2

            lax.fori_loop(0, CH // L, step, 0)
            out_base = (cbase - base_vec) * CODE_W
            pltpu.sync_copy(rows_v, out_hbm.at[pl.ds(out_base, CH * CODE_W)])
            return carry

        lax.fori_loop(0, n_chunks, chunk, 0)

    return body


def _sc_gather(cb_words, indices, base_vec):
    mesh = plsc.VectorSubcoreMesh(core_axis_name="c", subcore_axis_name="s")
    fn = pl.kernel(
        _make_sc_gather_body(base_vec),
        out_type=jax.ShapeDtypeStruct((VECS_PER_CALL * CODE_W,), jnp.int32),
        mesh=mesh,
        scratch_types=[
            pltpu.VMEM((NUM_CODES * CODE_W,), jnp.int32),
            pltpu.VMEM((CH,), jnp.int32),
            pltpu.VMEM((CH * CODE_W,), jnp.int32),
        ],
        compiler_params=pltpu.CompilerParams(needs_layout_passes=False),
    )
    return fn(cb_words, indices)


BM, BK = 2048, 1024
BN = ROWS_PER_CALL


def _mm_compute(x_ref, g_ref, s_ref, o_ref):
    k = pl.program_id(1)

    @pl.when(k == 0)
    def _zero():
        o_ref[...] = jnp.zeros_like(o_ref)

    o_ref[...] += lax.dot_general(
        x_ref[...], g_ref[...], (((1,), (1,)), ((), ())),
        preferred_element_type=jnp.float32,
        precision=lax.Precision.DEFAULT,
    )

    @pl.when(k == pl.num_programs(1) - 1)
    def _scale():
        o_ref[...] *= s_ref[...]


def _mm_body_first(x_ref, g_ref, s_ref, o_ref):
    _mm_compute(x_ref, g_ref, s_ref, o_ref)


def _mm_body_rest(x_ref, g_ref, s_ref, prev_ref, o_ref):
    _mm_compute(x_ref, g_ref, s_ref, o_ref)


def _tc_matmul_part(x, g, s_row, prev, col):
    m, kdim = x.shape
    grid = (m // BM, kdim // BK)
    in_specs = [
        pl.BlockSpec((BM, BK), lambda i, k: (i, k)),
        pl.BlockSpec((BN, BK), lambda i, k: (0, k)),
        pl.BlockSpec((1, BN), lambda i, k: (0, 0)),
    ]
    args = (x, g, s_row)
    if prev is None:
        body, aliases = _mm_body_first, {}
    else:
        body, aliases = _mm_body_rest, {3: 0}
        in_specs.append(pl.BlockSpec(memory_space=pl.ANY))
        args = args + (prev,)
    return pl.pallas_call(
        body,
        grid=grid,
        in_specs=in_specs,
        out_specs=pl.BlockSpec((BM, BN), lambda i, k, c=col: (i, c)),
        out_shape=jax.ShapeDtypeStruct((m, WEIGHT_ROWS), jnp.float32),
        input_output_aliases=aliases,
        compiler_params=pltpu.CompilerParams(
            dimension_semantics=("parallel", "arbitrary"),
            vmem_limit_bytes=100 * 1024 * 1024,
        ),
    )(*args)


def kernel(x, indices, codebook, scales):
    cb_words = lax.bitcast_convert_type(
        codebook.astype(jnp.bfloat16).reshape(NUM_CODES, CODE_W, 2), jnp.int32
    ).reshape(-1)
    x_bf = x.astype(jnp.bfloat16)
    s_row = scales.reshape(1, WEIGHT_ROWS)
    out = None
    for part in range(N_SPLIT):
        gw = _sc_gather(cb_words, indices, part * VECS_PER_CALL)
        g = lax.bitcast_convert_type(gw, jnp.bfloat16).reshape(
            ROWS_PER_CALL, WEIGHT_COLS
        )
        s_part = lax.slice_in_dim(s_row, part * BN, (part + 1) * BN, axis=1)
        out = _tc_matmul_part(x_bf, g, s_part, out, part)
    return out
